# baseline (speedup 1.0000x reference)
import functools
import math

import jax
import jax.numpy as jnp
from jax.experimental import pallas as pl
from jax.experimental.pallas import tpu as pltpu


def _gate_kernel(x_ref, w_ref, laux_ref, cw_ref, dm_ref, counts_ref, me_ref,
                 *, nb, bt, tokens, experts, cap):
    i = pl.program_id(0)

    @pl.when(i == 0)
    def _init():
        counts_ref[...] = jnp.zeros_like(counts_ref)
        me_ref[...] = jnp.zeros_like(me_ref)

    x = x_ref[...]
    w = w_ref[...]
    logits = jax.lax.dot_general(
        x, w, (((1,), (1,)), ((), ())), preferred_element_type=jnp.float32)

    mv = jnp.max(logits, axis=1, keepdims=True)
    ex = jnp.exp(logits - mv)
    gates = ex / jnp.sum(ex, axis=1, keepdims=True)

    e_idx = jax.lax.broadcasted_iota(jnp.int32, (bt, experts), 1)
    is_max = logits == mv
    first = jnp.min(jnp.where(is_max, e_idx, experts), axis=1, keepdims=True)
    mask = (e_idx == first).astype(jnp.float32)

    r = jax.lax.broadcasted_iota(jnp.int32, (bt, bt), 0)
    c = jax.lax.broadcasted_iota(jnp.int32, (bt, bt), 1)
    tri = (r >= c).astype(jnp.float32)
    csum = jax.lax.dot_general(
        tri, mask, (((1,), (0,)), ((), ())), preferred_element_type=jnp.float32)
    locations = csum - 1.0 + counts_ref[...]
    counts_ref[...] = counts_ref[...] + jnp.sum(mask, axis=0, keepdims=True)
    me_ref[...] = me_ref[...] + jnp.sum(gates, axis=0, keepdims=True)

    keep = mask * (locations < float(cap)).astype(jnp.float32)
    loc_s = jnp.sum(locations * keep, axis=1, keepdims=True)
    gate_s = jnp.sum(gates * keep, axis=1, keepdims=True)
    gates1 = gate_s * keep

    c_idx = jax.lax.broadcasted_iota(jnp.int32, (bt, cap), 1)
    onehot_c = (c_idx == loc_s.astype(jnp.int32)).astype(jnp.float32)
    cw = gates1[:, :, None] * onehot_c[:, None, :]
    cw_ref[...] = cw
    dm_ref[...] = cw != 0.0

    @pl.when(i == nb - 1)
    def _finish():
        me = me_ref[...] / float(tokens)
        ce = counts_ref[...] / float(tokens)
        laux_ref[0, 0] = jnp.sum(me * ce) * float(experts)


def kernel(input, wg):
    tokens, dim = input.shape
    experts = wg.shape[0]
    cap = math.ceil(tokens / experts)
    bt = 512
    nb = tokens // bt

    laux, cw, dm = pl.pallas_call(
        functools.partial(_gate_kernel, nb=nb, bt=bt, tokens=tokens,
                          experts=experts, cap=cap),
        grid=(nb,),
        in_specs=[
            pl.BlockSpec((bt, dim), lambda i: (i, 0)),
            pl.BlockSpec((experts, dim), lambda i: (0, 0)),
        ],
        out_specs=[
            pl.BlockSpec(memory_space=pltpu.SMEM),
            pl.BlockSpec((bt, experts, cap), lambda i: (i, 0, 0)),
            pl.BlockSpec((bt, experts, cap), lambda i: (i, 0, 0)),
        ],
        out_shape=[
            jax.ShapeDtypeStruct((1, 1), jnp.float32),
            jax.ShapeDtypeStruct((tokens, experts, cap), jnp.float32),
            jax.ShapeDtypeStruct((tokens, experts, cap), jnp.bool_),
        ],
        scratch_shapes=[
            pltpu.VMEM((1, experts), jnp.float32),
            pltpu.VMEM((1, experts), jnp.float32),
        ],
    )(input, wg)
    return laux[0, 0], cw, dm
